# baseline (speedup 1.0000x reference)
import functools

import jax
import jax.numpy as jnp
from jax import lax
from jax.experimental import pallas as pl
from jax.experimental.pallas import tpu as pltpu

B, G, D = 16, 2048, 128
NB = 8
NH = 4
HB = NB // NH


def _tc_body(phone_ref, *refs):
    adj_refs = refs[:NB]
    h_refs = refs[NB:NB + NH]
    obf_ref, oaf_ref = refs[NB + NH:]
    i = pl.program_id(0)

    for bl in range(NB):
        p = phone_ref[NB * i + bl]
        off = p % 8
        grp = adj_refs[bl][...]
        rsel = lax.broadcasted_iota(jnp.int32, (8, G), 0)
        row = jnp.sum(jnp.where(rsel == off, grp, 0.0), axis=0, keepdims=True)
        j = lax.broadcasted_iota(jnp.int32, (1, G), 1)
        wbf = jnp.where(j < p, row, 0.0)
        waf = jnp.where(j > p, row, 0.0)
        w = jnp.concatenate([wbf, waf], axis=0)
        hmat = h_refs[bl // HB][bl % HB]
        r = lax.dot_general(w, hmat, (((1,), (0,)), ((), ())),
                            preferred_element_type=jnp.float32)
        obf_ref[bl, 0] = r[0]
        oaf_ref[bl, 0] = r[1]


def _adj_index_map(bl, i, ph):
    return ((NB * i + bl) * (G // 8) + ph[NB * i + bl] // 8, 0)


def _h_index_map(j, i, ph):
    return (i * NH + j, 0, 0)


def kernel(h, adj_c, phone_set):
    phone = phone_set.astype(jnp.int32)

    grid_spec = pltpu.PrefetchScalarGridSpec(
        num_scalar_prefetch=1,
        grid=(B // NB,),
        in_specs=[
            pl.BlockSpec((8, G), functools.partial(_adj_index_map, bl))
            for bl in range(NB)
        ] + [
            pl.BlockSpec((HB, G, D), functools.partial(_h_index_map, j))
            for j in range(NH)
        ],
        out_specs=[
            pl.BlockSpec((NB, 1, D), lambda i, ph: (i, 0, 0)),
            pl.BlockSpec((NB, 1, D), lambda i, ph: (i, 0, 0)),
        ],
    )
    call = pl.pallas_call(
        _tc_body,
        grid_spec=grid_spec,
        compiler_params=pltpu.CompilerParams(
            dimension_semantics=("parallel",)),
        out_shape=(
            jax.ShapeDtypeStruct((B, 1, D), jnp.float32),
            jax.ShapeDtypeStruct((B, 1, D), jnp.float32),
        ),
    )
    adj2 = adj_c.reshape(B * G, G)
    g_bf, g_af = call(phone, *([adj2] * NB), *([h] * NH))
    return (g_bf.reshape(B, D), g_af.reshape(B, D))
